# baseline (speedup 1.0000x reference)
import functools

import jax
import jax.numpy as jnp
from jax import lax
from jax.experimental import pallas as pl
from jax.experimental.pallas import tpu as pltpu
from jax.experimental.pallas import tpu_sc as plsc

BATCH = 16384
EMBED = 32
NC = 2
NS = 16
NW = NC * NS
BPW = BATCH // NW
CHUNK = 128
NCHUNK = BPW // CHUNK

@functools.lru_cache(maxsize=1)
def _make_sc_gather():
    mesh = plsc.VectorSubcoreMesh(core_axis_name="c", subcore_axis_name="s")

    @functools.partial(
        pl.kernel,
        mesh=mesh,
        compiler_params=pltpu.CompilerParams(use_tc_tiling_on_sc=False),
        out_type=[
            jax.ShapeDtypeStruct((BATCH, EMBED), jnp.float32),
            jax.ShapeDtypeStruct((BATCH, EMBED), jnp.float32),
        ],
        scratch_types=[
            pltpu.VMEM((BPW,), jnp.int32),
            pltpu.VMEM((BPW,), jnp.int32),
            pltpu.VMEM((BPW, EMBED), jnp.float32),
            pltpu.VMEM((BPW, EMBED), jnp.float32),
            pltpu.SemaphoreType.DMA,
        ],
    )
    def _sc_gather(uidx_hbm, iidx_hbm, uemb_hbm, iemb_hbm, ue_out, ie_out,
                   uidx_v, iidx_v, urows_v, irows_v, sem):
        wid = lax.axis_index("s") * NC + lax.axis_index("c")
        base = wid * BPW
        pltpu.sync_copy(uidx_hbm.at[pl.ds(base, BPW)], uidx_v)
        pltpu.sync_copy(iidx_hbm.at[pl.ds(base, BPW)], iidx_v)
        copies = []
        for j in range(NCHUNK):
            sl = pl.ds(j * CHUNK, CHUNK)
            copies.append(pltpu.async_copy(uemb_hbm.at[uidx_v.at[sl]], urows_v.at[sl], sem))
            copies.append(pltpu.async_copy(iemb_hbm.at[iidx_v.at[sl]], irows_v.at[sl], sem))
        for c in copies:
            c.wait()
        pltpu.sync_copy(urows_v, ue_out.at[pl.ds(base, BPW)])
        pltpu.sync_copy(irows_v, ie_out.at[pl.ds(base, BPW)])

    return _sc_gather


def _mlp_body(ue_ref, ie_ref, w1u_ref, w1i_ref, b1_ref, w2_ref, b2_ref,
              wo_ref, bo_ref, out_ref):
    h = jnp.dot(ue_ref[...], w1u_ref[...], preferred_element_type=jnp.float32)
    h += jnp.dot(ie_ref[...], w1i_ref[...], preferred_element_type=jnp.float32)
    h = jnp.maximum(h + b1_ref[...], 0.0)
    h = jnp.dot(h, w2_ref[...], preferred_element_type=jnp.float32)
    h = jnp.maximum(h + b2_ref[...], 0.0)
    out_ref[...] = jnp.sum(h * wo_ref[...], axis=1, keepdims=True) + bo_ref[...]


def kernel(user_indices, item_indices, user_emb, item_emb, W1, b1, W2, b2, Wo, bo):
    ue, ie = _make_sc_gather()(user_indices.astype(jnp.int32),
                               item_indices.astype(jnp.int32),
                               user_emb, item_emb)
    w1t = W1.T
    out = pl.pallas_call(
        _mlp_body,
        out_shape=jax.ShapeDtypeStruct((BATCH, 1), jnp.float32),
    )(ue, ie, w1t[:EMBED], w1t[EMBED:], b1.reshape(1, -1),
      W2.T, b2.reshape(1, -1), Wo, bo.reshape(1, 1))
    return out
